# baseline (device time: 109606 ns/iter reference)
import os

import jax
import jax.numpy as jnp
from jax import lax
from jax.experimental import pallas as pl
from jax.experimental.pallas import tpu as pltpu

ABLATE_AR = os.environ.get("ABLATE_AR") == "1"
ABLATE_ATTN = os.environ.get("ABLATE_ATTN") == "1"

N_DEV = 4
B, SQ, SKV_G, HQ_G, DH = 2, 512, 2048, 32, 64
H_LOC = HQ_G // N_DEV
SKV_LOC = SKV_G // N_DEV
DM = 768
DQ_LOC = H_LOC * DH
QTR = SQ // N_DEV


def _body(x_ref, wq_ref, k_ref, v_ref, k8_ref, v8_ref, wo_ref, out_ref,
          kg_ref, vg_ref, kg8_ref, vg8_ref, myp_ref, rs_ref, ag_ref,
          q_ref, ctx_ref, send_sems, recv_sems):
    me = lax.axis_index("i")

    barrier = pltpu.get_barrier_semaphore()
    for s in range(1, N_DEV):
        pl.semaphore_signal(
            barrier, inc=1,
            device_id=((me + s) % N_DEV,),
            device_id_type=pl.DeviceIdType.MESH,
        )
    pl.semaphore_wait(barrier, N_DEV - 1)

    kg_ref[0] = k_ref[:, :, pl.ds(me * DQ_LOC, DQ_LOC)]
    vg_ref[0] = v_ref[:, :, pl.ds(me * DQ_LOC, DQ_LOC)]

    @pl.when(me >= 2)
    def _():
        kg8_ref[pl.ds(me - 2, 1)] = k8_ref[:, :, :, pl.ds(me * DQ_LOC, DQ_LOC)]
        vg8_ref[pl.ds(me - 2, 1)] = v8_ref[:, :, :, pl.ds(me * DQ_LOC, DQ_LOC)]

    copies = []
    copies8 = []
    for s in range(1, N_DEV):
        peer = (me + s) % N_DEV
        kcp = pltpu.make_async_remote_copy(
            src_ref=k_ref.at[:, :, pl.ds(peer * DQ_LOC, DQ_LOC)],
            dst_ref=kg_ref.at[N_DEV - s],
            send_sem=send_sems.at[s - 1],
            recv_sem=recv_sems.at[s - 1],
            device_id=(peer,),
            device_id_type=pl.DeviceIdType.MESH,
        )
        vcp = pltpu.make_async_remote_copy(
            src_ref=v_ref.at[:, :, pl.ds(peer * DQ_LOC, DQ_LOC)],
            dst_ref=vg_ref.at[N_DEV - s],
            send_sem=send_sems.at[3 + s - 1],
            recv_sem=recv_sems.at[3 + s - 1],
            device_id=(peer,),
            device_id_type=pl.DeviceIdType.MESH,
        )
        copies.append(kcp)
        copies.append(vcp)
        k8cp = pltpu.make_async_remote_copy(
            src_ref=k8_ref.at[:, :, :, pl.ds(peer * DQ_LOC, DQ_LOC)],
            dst_ref=kg8_ref.at[pl.ds((me - 2) % 2, 1)],
            send_sem=send_sems.at[s - 1],
            recv_sem=recv_sems.at[s - 1],
            device_id=(peer,),
            device_id_type=pl.DeviceIdType.MESH,
        )
        v8cp = pltpu.make_async_remote_copy(
            src_ref=v8_ref.at[:, :, :, pl.ds(peer * DQ_LOC, DQ_LOC)],
            dst_ref=vg8_ref.at[pl.ds((me - 2) % 2, 1)],
            send_sem=send_sems.at[3 + s - 1],
            recv_sem=recv_sems.at[3 + s - 1],
            device_id=(peer,),
            device_id_type=pl.DeviceIdType.MESH,
        )
        copies8.append(k8cp)
        copies8.append(v8cp)

    @pl.when(me <= 1)
    def _():
        for cp in copies:
            cp.start()

    @pl.when(me >= 2)
    def _():
        for cp in copies8:
            cp.start()

    x2d = x_ref[...].reshape(B * SQ, DM)
    qf = lax.dot_general(
        x2d, wq_ref[...], (((1,), (0,)), ((), ())),
        preferred_element_type=jnp.float32,
    )
    q_ref[...] = (qf * 0.125).astype(jnp.bfloat16)

    NA = SKV_LOC + 128
    NB = SKV_G - NA
    GR = 32
    qi = lax.broadcasted_iota(jnp.int32, (SQ, NA), 0)
    ki = lax.broadcasted_iota(jnp.int32, (SQ, NA), 1)
    mask_a = (jnp.abs(qi - ki) <= 128) | (ki < 32) | (qi < 32)
    mask_a0 = mask_a[:, :SKV_LOC]
    mask_a1 = mask_a[:, SKV_LOC:]
    neg = jnp.float32(-1e9)
    minf = jnp.full((SQ - GR, 1), -1e30, jnp.float32)
    zcol = jnp.zeros((SQ - GR, 1), jnp.float32)
    zctx = jnp.zeros((SQ - GR, DH), jnp.float32)

    for s in range(1, N_DEV):
        @pl.when((me - s) % N_DEV <= 1)
        def _(s=s):
            copies[2 * (s - 1)].wait_recv()
            copies[2 * (s - 1) + 1].wait_recv()


    slot = [(jnp.int32(c) - me) % N_DEV for c in range(N_DEV)]

    def dot_t(a, bm):
        return lax.dot_general(a, bm, (((1,), (1,)), ((), ())),
                               preferred_element_type=jnp.float32)

    def dot_n(a, bm):
        return lax.dot_general(a, bm, (((1,), (0,)), ((), ())),
                               preferred_element_type=jnp.float32)

    rs_copies = []
    for b in range(B):
        for h in range(H_LOC):
            if ABLATE_ATTN:
                break
            hs = slice(h * DH, (h + 1) * DH)
            qh = q_ref[pl.ds(b * SQ, SQ), hs]
            kc0 = kg_ref[pl.ds(slot[0], 1), b, :, hs][0]
            kc1 = kg_ref[pl.ds(slot[1], 1), b, :128, hs][0]
            vc0 = vg_ref[pl.ds(slot[0], 1), b, :, hs][0]
            vc1 = vg_ref[pl.ds(slot[1], 1), b, :128, hs][0]
            s0 = jnp.where(mask_a0, dot_t(qh, kc0), neg)
            s1 = jnp.where(mask_a1, dot_t(qh, kc1), neg)
            ma = jnp.maximum(jnp.max(s0, axis=1, keepdims=True),
                             jnp.max(s1, axis=1, keepdims=True))
            e0 = jnp.exp(s0 - ma).astype(jnp.bfloat16)
            e1 = jnp.exp(s1 - ma).astype(jnp.bfloat16)
            denom = (jnp.sum(e0.astype(jnp.float32), axis=1, keepdims=True)
                     + jnp.sum(e1.astype(jnp.float32), axis=1, keepdims=True))
            na = dot_n(e0, vc0) + dot_n(e1, vc1)
            ctx_ref[pl.ds(b * SQ, SQ), hs] = (na / denom).astype(jnp.bfloat16)

    for s in range(1, N_DEV):
        @pl.when((me - s) % N_DEV >= 2)
        def _(s=s):
            copies8[2 * (s - 1)].wait_recv()
            copies8[2 * (s - 1) + 1].wait_recv()

    for b in range(B):
        for h in range(H_LOC):
            if ABLATE_ATTN:
                break
            hs = slice(h * DH, (h + 1) * DH)
            q32 = q_ref[pl.ds(b * SQ, GR), hs]
            kfull = jnp.concatenate(
                [kg_ref[pl.ds(slot[0], 1), b, :, hs][0],
                 kg_ref[pl.ds(slot[1], 1), b, :, hs][0],
                 kg8_ref[0, b, :, hs].astype(jnp.bfloat16),
                 kg8_ref[1, b, :, hs].astype(jnp.bfloat16)], axis=0)
            vfull = jnp.concatenate(
                [vg_ref[pl.ds(slot[0], 1), b, :, hs][0],
                 vg_ref[pl.ds(slot[1], 1), b, :, hs][0],
                 vg8_ref[0, b, :, hs].astype(jnp.bfloat16),
                 vg8_ref[1, b, :, hs].astype(jnp.bfloat16)], axis=0)
            sfull = dot_t(q32, kfull)
            m32 = jnp.max(sfull, axis=1, keepdims=True)
            efull = jnp.exp(sfull - m32).astype(jnp.bfloat16)
            den32 = jnp.sum(efull.astype(jnp.float32), axis=1, keepdims=True)
            num32 = dot_n(efull, vfull)
            ctx_ref[pl.ds(b * SQ, GR), hs] = (num32 / den32).astype(jnp.bfloat16)

        bsl = pl.ds(b * SQ, SQ)
        partial_b = lax.dot_general(
            ctx_ref[bsl], wo_ref[...], (((1,), (0,)), ((), ())),
            preferred_element_type=jnp.float32,
        )
        myp_ref[bsl] = partial_b.astype(jnp.bfloat16)
        if not ABLATE_AR:
            for s in range(1, N_DEV):
                peer = (me + s) % N_DEV
                cp = pltpu.make_async_remote_copy(
                    src_ref=myp_ref.at[pl.ds(b * SQ + peer * QTR, QTR)],
                    dst_ref=rs_ref.at[b, s - 1],
                    send_sem=send_sems.at[6 + 3 * b + s - 1],
                    recv_sem=recv_sems.at[6 + 3 * b + s - 1],
                    device_id=(peer,),
                    device_id_type=pl.DeviceIdType.MESH,
                )
                cp.start()
                rs_copies.append(cp)

    if ABLATE_AR:
        out_ref[...] = myp_ref[...].astype(jnp.float32).reshape(B, SQ, DM)
        return

    ag_copies = []
    for b in range(B):
        for t in range(3):
            rs_copies[3 * b + t].wait_recv()
        myq = myp_ref[pl.ds(b * SQ + me * QTR, QTR)].astype(jnp.float32)
        for t in range(3):
            myq = myq + rs_ref[b, t].astype(jnp.float32)
        ag_ref[b, 3] = myq.astype(jnp.bfloat16)
        out_ref[b, pl.ds(me * QTR, QTR)] = myq
        for s in range(1, N_DEV):
            peer = (me + s) % N_DEV
            cp = pltpu.make_async_remote_copy(
                src_ref=ag_ref.at[b, 3],
                dst_ref=ag_ref.at[b, s - 1],
                send_sem=send_sems.at[12 + 3 * b + s - 1],
                recv_sem=recv_sems.at[12 + 3 * b + s - 1],
                device_id=(peer,),
                device_id_type=pl.DeviceIdType.MESH,
            )
            cp.start()
            ag_copies.append(cp)

    for b in range(B):
        for s in range(1, N_DEV):
            ag_copies[3 * b + s - 1].wait_recv()
            origin = (me - s) % N_DEV
            out_ref[b, pl.ds(origin * QTR, QTR)] = (
                ag_ref[b, s - 1].astype(jnp.float32))

    @pl.when(me <= 1)
    def _():
        for cp in copies:
            cp.wait_send()

    @pl.when(me >= 2)
    def _():
        for cp in copies8:
            cp.wait_send()

    for cp in rs_copies + ag_copies:
        cp.wait_send()


def kernel(x, Wq, K_ext, V_ext, Wo):
    xb = x.astype(jnp.bfloat16)
    wqb = Wq.astype(jnp.bfloat16)
    wob = Wo.astype(jnp.bfloat16)
    kb = K_ext.reshape(B, SKV_LOC, HQ_G * DH).astype(jnp.bfloat16)
    vb = V_ext.reshape(B, SKV_LOC, HQ_G * DH).astype(jnp.bfloat16)
    k8 = kb.astype(jnp.float8_e4m3fn)[None]
    v8 = vb.astype(jnp.float8_e4m3fn)[None]

    out_shape = jax.ShapeDtypeStruct((B, SQ, DM), jnp.float32)
    return pl.pallas_call(
        _body,
        out_shape=out_shape,
        in_specs=[pl.BlockSpec(memory_space=pltpu.VMEM)] * 7,
        out_specs=pl.BlockSpec(memory_space=pltpu.VMEM),
        scratch_shapes=[
            pltpu.VMEM((N_DEV, B, SKV_LOC, DQ_LOC), jnp.bfloat16),
            pltpu.VMEM((N_DEV, B, SKV_LOC, DQ_LOC), jnp.bfloat16),
            pltpu.VMEM((2, B, SKV_LOC, DQ_LOC), jnp.float8_e4m3fn),
            pltpu.VMEM((2, B, SKV_LOC, DQ_LOC), jnp.float8_e4m3fn),
            pltpu.VMEM((B * SQ, DM), jnp.bfloat16),
            pltpu.VMEM((B, 3, QTR, DM), jnp.bfloat16),
            pltpu.VMEM((B, 4, QTR, DM), jnp.bfloat16),
            pltpu.VMEM((B * SQ, DQ_LOC), jnp.bfloat16),
            pltpu.VMEM((B * SQ, DQ_LOC), jnp.bfloat16),
            pltpu.SemaphoreType.DMA((18,)),
            pltpu.SemaphoreType.DMA((18,)),
        ],
        compiler_params=pltpu.CompilerParams(collective_id=0),
    )(xb, wqb, kb, vb, k8, v8, wob)


# device time: 98041 ns/iter; 1.1180x vs baseline; 1.1180x over previous
import os

import jax
import jax.numpy as jnp
from jax import lax
from jax.experimental import pallas as pl
from jax.experimental.pallas import tpu as pltpu

ABLATE_AR = os.environ.get("ABLATE_AR") == "1"
ABLATE_ATTN = os.environ.get("ABLATE_ATTN") == "1"

N_DEV = 4
B, SQ, SKV_G, HQ_G, DH = 2, 512, 2048, 32, 64
H_LOC = HQ_G // N_DEV
SKV_LOC = SKV_G // N_DEV
DM = 768
DQ_LOC = H_LOC * DH
QTR = SQ // N_DEV


def _body(x_ref, wq_ref, k_ref, v_ref, wo_ref, out_ref,
          kg_ref, vg_ref, myp_ref, rs_ref, ag_ref, q_ref, ctx_ref,
          send_sems, recv_sems):
    me = lax.axis_index("i")

    barrier = pltpu.get_barrier_semaphore()
    for s in range(1, N_DEV):
        pl.semaphore_signal(
            barrier, inc=1,
            device_id=((me + s) % N_DEV,),
            device_id_type=pl.DeviceIdType.MESH,
        )
    pl.semaphore_wait(barrier, N_DEV - 1)

    kg_ref[0] = k_ref[:, :, pl.ds(me * DQ_LOC, DQ_LOC)]
    vg_ref[0] = v_ref[:, :, pl.ds(me * DQ_LOC, DQ_LOC)]

    copies = []
    for s in range(1, N_DEV):
        peer = (me + s) % N_DEV
        kcp = pltpu.make_async_remote_copy(
            src_ref=k_ref.at[:, :, pl.ds(peer * DQ_LOC, DQ_LOC)],
            dst_ref=kg_ref.at[N_DEV - s],
            send_sem=send_sems.at[s - 1],
            recv_sem=recv_sems.at[s - 1],
            device_id=(peer,),
            device_id_type=pl.DeviceIdType.MESH,
        )
        vcp = pltpu.make_async_remote_copy(
            src_ref=v_ref.at[:, :, pl.ds(peer * DQ_LOC, DQ_LOC)],
            dst_ref=vg_ref.at[N_DEV - s],
            send_sem=send_sems.at[3 + s - 1],
            recv_sem=recv_sems.at[3 + s - 1],
            device_id=(peer,),
            device_id_type=pl.DeviceIdType.MESH,
        )
        copies.append(kcp)
        copies.append(vcp)

    for cp in copies:
        cp.start()

    x2d = x_ref[...].reshape(B * SQ, DM)
    qf = lax.dot_general(
        x2d, wq_ref[...], (((1,), (0,)), ((), ())),
        preferred_element_type=jnp.float32,
    )
    q_ref[...] = (qf * 0.125).astype(jnp.bfloat16)

    NA = SKV_LOC + 128
    NB = SKV_G - NA
    GR = 32
    qi = lax.broadcasted_iota(jnp.int32, (SQ, NA), 0)
    ki = lax.broadcasted_iota(jnp.int32, (SQ, NA), 1)
    mask_a = (jnp.abs(qi - ki) <= 128) | (ki < 32) | (qi < 32)
    mask_a0 = mask_a[:, :SKV_LOC]
    mask_a1 = mask_a[:, SKV_LOC:]
    neg = jnp.float32(-1e9)
    minf = jnp.full((SQ - GR, 1), -1e30, jnp.float32)
    zcol = jnp.zeros((SQ - GR, 1), jnp.float32)
    zctx = jnp.zeros((SQ - GR, DH), jnp.float32)

    for s in range(1, N_DEV):
        @pl.when((me - s) % N_DEV <= 1)
        def _(s=s):
            copies[2 * (s - 1)].wait_recv()
            copies[2 * (s - 1) + 1].wait_recv()


    slot = [(jnp.int32(c) - me) % N_DEV for c in range(N_DEV)]

    def dot_t(a, bm):
        return lax.dot_general(a, bm, (((1,), (1,)), ((), ())),
                               preferred_element_type=jnp.float32)

    def dot_n(a, bm):
        return lax.dot_general(a, bm, (((1,), (0,)), ((), ())),
                               preferred_element_type=jnp.float32)

    rs_copies = []
    for b in range(B):
        for h in range(H_LOC):
            if ABLATE_ATTN:
                break
            hs = slice(h * DH, (h + 1) * DH)
            qh = q_ref[pl.ds(b * SQ, SQ), hs]
            kc0 = kg_ref[pl.ds(slot[0], 1), b, :, hs][0]
            kc1 = kg_ref[pl.ds(slot[1], 1), b, :128, hs][0]
            vc0 = vg_ref[pl.ds(slot[0], 1), b, :, hs][0]
            vc1 = vg_ref[pl.ds(slot[1], 1), b, :128, hs][0]
            s0 = jnp.where(mask_a0, dot_t(qh, kc0), neg)
            s1 = jnp.where(mask_a1, dot_t(qh, kc1), neg)
            ma = jnp.maximum(jnp.max(s0, axis=1, keepdims=True),
                             jnp.max(s1, axis=1, keepdims=True))
            e0 = jnp.exp(s0 - ma).astype(jnp.bfloat16)
            e1 = jnp.exp(s1 - ma).astype(jnp.bfloat16)
            denom = (jnp.sum(e0.astype(jnp.float32), axis=1, keepdims=True)
                     + jnp.sum(e1.astype(jnp.float32), axis=1, keepdims=True))
            na = dot_n(e0, vc0) + dot_n(e1, vc1)
            ctx_ref[pl.ds(b * SQ, SQ), hs] = (na / denom).astype(jnp.bfloat16)

    for s in range(1, N_DEV):
        @pl.when((me - s) % N_DEV >= 2)
        def _(s=s):
            copies[2 * (s - 1)].wait_recv()
            copies[2 * (s - 1) + 1].wait_recv()

    for b in range(B):
        for h in range(H_LOC):
            if ABLATE_ATTN:
                break
            hs = slice(h * DH, (h + 1) * DH)
            q32 = q_ref[pl.ds(b * SQ, GR), hs]
            sc = [dot_t(q32, kg_ref[pl.ds(slot[c], 1), b, :, hs][0])
                  for c in range(N_DEV)]
            m32 = jnp.maximum(
                jnp.maximum(jnp.max(sc[0], axis=1, keepdims=True),
                            jnp.max(sc[1], axis=1, keepdims=True)),
                jnp.maximum(jnp.max(sc[2], axis=1, keepdims=True),
                            jnp.max(sc[3], axis=1, keepdims=True)))
            ec = [jnp.exp(s - m32).astype(jnp.bfloat16) for s in sc]
            den32 = (jnp.sum(ec[0].astype(jnp.float32), axis=1, keepdims=True)
                     + jnp.sum(ec[1].astype(jnp.float32), axis=1, keepdims=True)
                     + jnp.sum(ec[2].astype(jnp.float32), axis=1, keepdims=True)
                     + jnp.sum(ec[3].astype(jnp.float32), axis=1, keepdims=True))
            num32 = (dot_n(ec[0], vg_ref[pl.ds(slot[0], 1), b, :, hs][0])
                     + dot_n(ec[1], vg_ref[pl.ds(slot[1], 1), b, :, hs][0])
                     + dot_n(ec[2], vg_ref[pl.ds(slot[2], 1), b, :, hs][0])
                     + dot_n(ec[3], vg_ref[pl.ds(slot[3], 1), b, :, hs][0]))
            ctx_ref[pl.ds(b * SQ, GR), hs] = (num32 / den32).astype(jnp.bfloat16)

        bsl = pl.ds(b * SQ, SQ)
        partial_b = lax.dot_general(
            ctx_ref[bsl], wo_ref[...], (((1,), (0,)), ((), ())),
            preferred_element_type=jnp.float32,
        )
        myp_ref[bsl] = partial_b.astype(jnp.bfloat16)
        if not ABLATE_AR:
            for s in range(1, N_DEV):
                peer = (me + s) % N_DEV
                cp = pltpu.make_async_remote_copy(
                    src_ref=myp_ref.at[pl.ds(b * SQ + peer * QTR, QTR)],
                    dst_ref=rs_ref.at[b, s - 1],
                    send_sem=send_sems.at[6 + 3 * b + s - 1],
                    recv_sem=recv_sems.at[6 + 3 * b + s - 1],
                    device_id=(peer,),
                    device_id_type=pl.DeviceIdType.MESH,
                )
                cp.start()
                rs_copies.append(cp)

    if ABLATE_AR:
        out_ref[...] = myp_ref[...].astype(jnp.float32).reshape(B, SQ, DM)
        for cp in copies:
            cp.wait_send()
        return

    ag_copies = []
    for b in range(B):
        for t in range(3):
            rs_copies[3 * b + t].wait_recv()
        myq = myp_ref[pl.ds(b * SQ + me * QTR, QTR)].astype(jnp.float32)
        for t in range(3):
            myq = myq + rs_ref[b, t].astype(jnp.float32)
        ag_ref[b, 3] = myq.astype(jnp.bfloat16)
        out_ref[b, pl.ds(me * QTR, QTR)] = myq
        for s in range(1, N_DEV):
            peer = (me + s) % N_DEV
            cp = pltpu.make_async_remote_copy(
                src_ref=ag_ref.at[b, 3],
                dst_ref=ag_ref.at[b, s - 1],
                send_sem=send_sems.at[12 + 3 * b + s - 1],
                recv_sem=recv_sems.at[12 + 3 * b + s - 1],
                device_id=(peer,),
                device_id_type=pl.DeviceIdType.MESH,
            )
            cp.start()
            ag_copies.append(cp)

    for b in range(B):
        for s in range(1, N_DEV):
            ag_copies[3 * b + s - 1].wait_recv()
            origin = (me - s) % N_DEV
            out_ref[b, pl.ds(origin * QTR, QTR)] = (
                ag_ref[b, s - 1].astype(jnp.float32))

    for cp in copies:
        cp.wait_send()
    for cp in rs_copies + ag_copies:
        cp.wait_send()


def kernel(x, Wq, K_ext, V_ext, Wo):
    xb = x.astype(jnp.bfloat16)
    wqb = Wq.astype(jnp.bfloat16)
    wob = Wo.astype(jnp.bfloat16)
    kb = K_ext.reshape(B, SKV_LOC, HQ_G * DH).astype(jnp.bfloat16)
    vb = V_ext.reshape(B, SKV_LOC, HQ_G * DH).astype(jnp.bfloat16)

    out_shape = jax.ShapeDtypeStruct((B, SQ, DM), jnp.float32)
    return pl.pallas_call(
        _body,
        out_shape=out_shape,
        in_specs=[pl.BlockSpec(memory_space=pltpu.VMEM)] * 5,
        out_specs=pl.BlockSpec(memory_space=pltpu.VMEM),
        scratch_shapes=[
            pltpu.VMEM((N_DEV, B, SKV_LOC, DQ_LOC), jnp.bfloat16),
            pltpu.VMEM((N_DEV, B, SKV_LOC, DQ_LOC), jnp.bfloat16),
            pltpu.VMEM((B * SQ, DM), jnp.bfloat16),
            pltpu.VMEM((B, 3, QTR, DM), jnp.bfloat16),
            pltpu.VMEM((B, 4, QTR, DM), jnp.bfloat16),
            pltpu.VMEM((B * SQ, DQ_LOC), jnp.bfloat16),
            pltpu.VMEM((B * SQ, DQ_LOC), jnp.bfloat16),
            pltpu.SemaphoreType.DMA((18,)),
            pltpu.SemaphoreType.DMA((18,)),
        ],
        compiler_params=pltpu.CompilerParams(collective_id=0),
    )(xb, wqb, kb, vb, wob)
